# initial kernel scaffold (unmeasured)
import jax
import jax.numpy as jnp
from jax import lax
from jax.experimental import pallas as pl
from jax.experimental.pallas import tpu as pltpu

N_DEV = 16
N_STEPS = 4
N_LAYERS = 3
N_EXCH = N_LAYERS * N_STEPS


def kernel(x, Win0, Wout0, Win1, Wout1, Win2, Wout2):
    b, d = x.shape

    def body(x_ref, win0_ref, wout0_ref, win1_ref, wout1_ref, win2_ref,
             wout2_ref, out_ref, send_ref, recv_ref, send_sems, recv_sems):
        my = lax.axis_index("i")
        wins = [win0_ref, win1_ref, win2_ref]
        wouts = [wout0_ref, wout1_ref, wout2_ref]

        xb = x_ref[...].astype(jnp.bfloat16)
        acc = None
        for layer in range(N_LAYERS):
            w_in = wins[layer][...].astype(jnp.bfloat16)
            w_out = wouts[layer][...].astype(jnp.bfloat16)
            h = jnp.maximum(
                jnp.dot(xb, w_in, preferred_element_type=jnp.float32), 0.0
            ).astype(jnp.bfloat16)
            acc = jnp.dot(h, w_out, preferred_element_type=jnp.float32)

            for s in range(N_STEPS):
                idx = layer * N_STEPS + s
                partner = my ^ (1 << s)
                send_ref[...] = acc.astype(jnp.bfloat16)
                rdma = pltpu.make_async_remote_copy(
                    src_ref=send_ref,
                    dst_ref=recv_ref.at[idx],
                    send_sem=send_sems.at[idx],
                    recv_sem=recv_sems.at[idx],
                    device_id=(partner,),
                    device_id_type=pl.DeviceIdType.MESH,
                )
                rdma.start()
                rdma.wait()
                acc = acc + recv_ref[idx].astype(jnp.float32)
            xb = acc.astype(jnp.bfloat16)

        out_ref[...] = acc

    return pl.pallas_call(
        body,
        out_shape=jax.ShapeDtypeStruct((b, d), jnp.float32),
        in_specs=[pl.BlockSpec(memory_space=pltpu.VMEM)] * 7,
        out_specs=pl.BlockSpec(memory_space=pltpu.VMEM),
        scratch_shapes=[
            pltpu.VMEM((b, d), jnp.bfloat16),
            pltpu.VMEM((N_EXCH, b, d), jnp.bfloat16),
            pltpu.SemaphoreType.DMA((N_EXCH,)),
            pltpu.SemaphoreType.DMA((N_EXCH,)),
        ],
        compiler_params=pltpu.CompilerParams(collective_id=0),
    )(x, Win0, Wout0, Win1, Wout1, Win2, Wout2)


# baseline (device time: 84475 ns/iter reference)
import jax
import jax.numpy as jnp
from jax import lax
from jax.experimental import pallas as pl
from jax.experimental.pallas import tpu as pltpu

N_DEV = 16
N_STEPS = 4
N_LAYERS = 3
N_EXCH = N_LAYERS * N_STEPS


def kernel(x, Win0, Wout0, Win1, Wout1, Win2, Wout2):
    b, d = x.shape

    def body(x_ref, win0_ref, wout0_ref, win1_ref, wout1_ref, win2_ref,
             wout2_ref, out_ref, send_ref, recv_ref, send_sems, recv_sems):
        my = lax.axis_index("i")
        wins = [win0_ref, win1_ref, win2_ref]
        wouts = [wout0_ref, wout1_ref, wout2_ref]

        xb = x_ref[...].astype(jnp.bfloat16)
        acc = None
        for layer in range(N_LAYERS):
            w_in = wins[layer][...].astype(jnp.bfloat16)
            w_out = wouts[layer][...].astype(jnp.bfloat16)
            h = jnp.maximum(
                jnp.dot(xb, w_in, preferred_element_type=jnp.float32), 0.0
            ).astype(jnp.bfloat16)
            acc = jnp.dot(h, w_out, preferred_element_type=jnp.float32)

            for s in range(N_STEPS):
                idx = layer * N_STEPS + s
                partner = my ^ (1 << s)
                send_ref[...] = acc.astype(jnp.bfloat16)
                rdma = pltpu.make_async_remote_copy(
                    src_ref=send_ref,
                    dst_ref=recv_ref.at[idx],
                    send_sem=send_sems.at[idx],
                    recv_sem=recv_sems.at[idx],
                    device_id=(partner,),
                    device_id_type=pl.DeviceIdType.MESH,
                )
                rdma.start()
                rdma.wait()
                acc = acc + recv_ref[idx].astype(jnp.float32)
            xb = acc.astype(jnp.bfloat16)

        out_ref[...] = acc

    return pl.pallas_call(
        body,
        out_shape=jax.ShapeDtypeStruct((b, d), jnp.float32),
        in_specs=[pl.BlockSpec(memory_space=pltpu.VMEM)] * 7,
        out_specs=pl.BlockSpec(memory_space=pltpu.VMEM),
        scratch_shapes=[
            pltpu.VMEM((b, d), jnp.bfloat16),
            pltpu.VMEM((N_EXCH, b, d), jnp.bfloat16),
            pltpu.SemaphoreType.DMA((N_EXCH,)),
            pltpu.SemaphoreType.DMA((N_EXCH,)),
        ],
    )(x, Win0, Wout0, Win1, Wout1, Win2, Wout2)


# device time: 50374 ns/iter; 1.6770x vs baseline; 1.6770x over previous
import jax
import jax.numpy as jnp
from jax import lax
from jax.experimental import pallas as pl
from jax.experimental.pallas import tpu as pltpu

N_DEV = 16
N_LAYERS = 3


def kernel(x, Win0, Wout0, Win1, Wout1, Win2, Wout2):
    b, d = x.shape
    ch = b // N_DEV

    def body(x_ref, win0_ref, wout0_ref, win1_ref, wout1_ref, win2_ref,
             wout2_ref, out_ref, send_ref, rs_recv, xbuf,
             rs_send_sems, rs_recv_sems, ag_send_sems, ag_recv_sems):
        my = lax.axis_index("i")
        wins = [win0_ref, win1_ref, win2_ref]
        wouts = [wout0_ref, wout1_ref, wout2_ref]

        xb = x_ref[...].astype(jnp.bfloat16)
        for layer in range(N_LAYERS):
            w_in = wins[layer][...].astype(jnp.bfloat16)
            w_out = wouts[layer][...].astype(jnp.bfloat16)
            h = jnp.maximum(
                jnp.dot(xb, w_in, preferred_element_type=jnp.float32), 0.0
            ).astype(jnp.bfloat16)
            acc = jnp.dot(h, w_out, preferred_element_type=jnp.float32)
            send_ref[...] = acc.astype(jnp.bfloat16)

            for k in range(N_DEV):
                rdma = pltpu.make_async_remote_copy(
                    src_ref=send_ref.at[pl.ds(k * ch, ch)],
                    dst_ref=rs_recv.at[my],
                    send_sem=rs_send_sems.at[k],
                    recv_sem=rs_recv_sems.at[my],
                    device_id=(k,),
                    device_id_type=pl.DeviceIdType.MESH,
                )

                @pl.when(my != k)
                def _():
                    rdma.start()

            local = pltpu.make_async_copy(
                send_ref.at[pl.ds(my * ch, ch)],
                rs_recv.at[my],
                rs_recv_sems.at[my],
            )
            local.start()

            for k in range(N_DEV):
                waiter = pltpu.make_async_remote_copy(
                    src_ref=send_ref.at[pl.ds(0, ch)],
                    dst_ref=rs_recv.at[k],
                    send_sem=rs_send_sems.at[k],
                    recv_sem=rs_recv_sems.at[k],
                    device_id=(k,),
                    device_id_type=pl.DeviceIdType.MESH,
                )
                waiter.wait_recv()

            reduced = jnp.sum(rs_recv[...].astype(jnp.float32), axis=0)

            xbuf[pl.ds(my * ch, ch), :] = reduced.astype(jnp.bfloat16)
            for k in range(N_DEV):
                rdma = pltpu.make_async_remote_copy(
                    src_ref=xbuf.at[pl.ds(my * ch, ch)],
                    dst_ref=xbuf.at[pl.ds(my * ch, ch)],
                    send_sem=ag_send_sems.at[k],
                    recv_sem=ag_recv_sems.at[my],
                    device_id=(k,),
                    device_id_type=pl.DeviceIdType.MESH,
                )

                @pl.when(my != k)
                def _():
                    rdma.start()

            for k in range(N_DEV):
                waiter = pltpu.make_async_remote_copy(
                    src_ref=xbuf.at[pl.ds(k * ch, ch)],
                    dst_ref=xbuf.at[pl.ds(k * ch, ch)],
                    send_sem=ag_send_sems.at[k],
                    recv_sem=ag_recv_sems.at[k],
                    device_id=(k,),
                    device_id_type=pl.DeviceIdType.MESH,
                )

                @pl.when(my != k)
                def _():
                    waiter.wait_recv()

            for k in range(N_DEV):
                rs_s = pltpu.make_async_remote_copy(
                    src_ref=send_ref.at[pl.ds(k * ch, ch)],
                    dst_ref=rs_recv.at[k],
                    send_sem=rs_send_sems.at[k],
                    recv_sem=rs_recv_sems.at[k],
                    device_id=(k,),
                    device_id_type=pl.DeviceIdType.MESH,
                )
                ag_s = pltpu.make_async_remote_copy(
                    src_ref=xbuf.at[pl.ds(my * ch, ch)],
                    dst_ref=xbuf.at[pl.ds(my * ch, ch)],
                    send_sem=ag_send_sems.at[k],
                    recv_sem=ag_recv_sems.at[k],
                    device_id=(k,),
                    device_id_type=pl.DeviceIdType.MESH,
                )

                @pl.when(my != k)
                def _():
                    rs_s.wait_send()
                    ag_s.wait_send()

            xb = xbuf[...]

        out_ref[...] = xbuf[...].astype(jnp.float32)

    return pl.pallas_call(
        body,
        out_shape=jax.ShapeDtypeStruct((b, d), jnp.float32),
        in_specs=[pl.BlockSpec(memory_space=pltpu.VMEM)] * 7,
        out_specs=pl.BlockSpec(memory_space=pltpu.VMEM),
        scratch_shapes=[
            pltpu.VMEM((b, d), jnp.bfloat16),
            pltpu.VMEM((N_DEV, ch, d), jnp.bfloat16),
            pltpu.VMEM((b, d), jnp.bfloat16),
            pltpu.SemaphoreType.DMA((N_DEV,)),
            pltpu.SemaphoreType.DMA((N_DEV,)),
            pltpu.SemaphoreType.DMA((N_DEV,)),
            pltpu.SemaphoreType.DMA((N_DEV,)),
        ],
    )(x, Win0, Wout0, Win1, Wout1, Win2, Wout2)


# device time: 49878 ns/iter; 1.6936x vs baseline; 1.0099x over previous
import jax
import jax.numpy as jnp
from jax import lax
from jax.experimental import pallas as pl
from jax.experimental.pallas import tpu as pltpu

N_DEV = 16
N_LAYERS = 3
STAGES = 4


def kernel(x, Win0, Wout0, Win1, Wout1, Win2, Wout2):
    b, d = x.shape
    ch = b // N_DEV
    rps = b // STAGES
    ops = N_DEV // STAGES

    def body(x_ref, win0_ref, wout0_ref, win1_ref, wout1_ref, win2_ref,
             wout2_ref, out_ref, send_ref, rs_recv, xbuf,
             rs_send_sems, rs_recv_sems, ag_send_sems, ag_recv_sems):
        my = lax.axis_index("i")
        wins = [win0_ref, win1_ref, win2_ref]
        wouts = [wout0_ref, wout1_ref, wout2_ref]

        def rs_descr(k):
            return pltpu.make_async_remote_copy(
                src_ref=send_ref.at[pl.ds(k * ch, ch)],
                dst_ref=rs_recv.at[my],
                send_sem=rs_send_sems.at[k],
                recv_sem=rs_recv_sems.at[my],
                device_id=(k,),
                device_id_type=pl.DeviceIdType.MESH,
            )

        def rs_waiter(k):
            return pltpu.make_async_remote_copy(
                src_ref=send_ref.at[pl.ds(0, ch)],
                dst_ref=rs_recv.at[k],
                send_sem=rs_send_sems.at[k],
                recv_sem=rs_recv_sems.at[k],
                device_id=(k,),
                device_id_type=pl.DeviceIdType.MESH,
            )

        def ag_descr(k, src_rows, sem_row):
            return pltpu.make_async_remote_copy(
                src_ref=xbuf.at[pl.ds(src_rows * ch, ch)],
                dst_ref=xbuf.at[pl.ds(src_rows * ch, ch)],
                send_sem=ag_send_sems.at[k],
                recv_sem=ag_recv_sems.at[sem_row],
                device_id=(k,),
                device_id_type=pl.DeviceIdType.MESH,
            )

        def ag_out_descr(k, src_rows, sem_row):
            return pltpu.make_async_remote_copy(
                src_ref=out_ref.at[pl.ds(src_rows * ch, ch)],
                dst_ref=out_ref.at[pl.ds(src_rows * ch, ch)],
                send_sem=ag_send_sems.at[k],
                recv_sem=ag_recv_sems.at[sem_row],
                device_id=(k,),
                device_id_type=pl.DeviceIdType.MESH,
            )

        for layer in range(N_LAYERS):
            w_in = wins[layer][...].astype(jnp.bfloat16)
            w_out = wouts[layer][...].astype(jnp.bfloat16)

            for s in range(STAGES):
                r0 = s * rps
                if layer == 0:
                    xh = x_ref[pl.ds(r0, rps), :].astype(jnp.bfloat16)
                else:
                    for k in range(s * ops, (s + 1) * ops):
                        waiter = ag_descr(k, k, k)

                        @pl.when(my != k)
                        def _():
                            waiter.wait_recv()

                    xh = xbuf[pl.ds(r0, rps), :]
                h = jnp.maximum(
                    jnp.dot(xh, w_in, preferred_element_type=jnp.float32), 0.0
                ).astype(jnp.bfloat16)
                p = jnp.dot(h, w_out, preferred_element_type=jnp.float32)
                send_ref[pl.ds(r0, rps), :] = p.astype(jnp.bfloat16)
                for k in range(s * ops, (s + 1) * ops):
                    rdma = rs_descr(k)

                    @pl.when(my != k)
                    def _():
                        rdma.start()

            local = pltpu.make_async_copy(
                send_ref.at[pl.ds(my * ch, ch)],
                rs_recv.at[my],
                rs_recv_sems.at[my],
            )
            local.start()

            for k in range(N_DEV):
                rs_waiter(k).wait_recv()

            reduced = jnp.sum(rs_recv[...].astype(jnp.float32), axis=0)

            if layer < N_LAYERS - 1:
                xbuf[pl.ds(my * ch, ch), :] = reduced.astype(jnp.bfloat16)
                for k in range(N_DEV):
                    rdma = ag_descr(k, my, my)

                    @pl.when(my != k)
                    def _():
                        rdma.start()
            else:
                out_ref[pl.ds(my * ch, ch), :] = reduced
                for k in range(N_DEV):
                    rdma = ag_out_descr(k, my, my)

                    @pl.when(my != k)
                    def _():
                        rdma.start()

                for k in range(N_DEV):
                    waiter = ag_out_descr(k, k, k)

                    @pl.when(my != k)
                    def _():
                        waiter.wait_recv()

            for k in range(N_DEV):
                rs_s = rs_waiter(k)
                ag_s = (ag_descr if layer < N_LAYERS - 1 else ag_out_descr)(
                    k, my, my
                )

                @pl.when(my != k)
                def _():
                    rs_s.wait_send()
                    ag_s.wait_send()

    return pl.pallas_call(
        body,
        out_shape=jax.ShapeDtypeStruct((b, d), jnp.float32),
        in_specs=[pl.BlockSpec(memory_space=pltpu.VMEM)] * 7,
        out_specs=pl.BlockSpec(memory_space=pltpu.VMEM),
        scratch_shapes=[
            pltpu.VMEM((b, d), jnp.bfloat16),
            pltpu.VMEM((N_DEV, ch, d), jnp.bfloat16),
            pltpu.VMEM((b, d), jnp.bfloat16),
            pltpu.SemaphoreType.DMA((N_DEV,)),
            pltpu.SemaphoreType.DMA((N_DEV,)),
            pltpu.SemaphoreType.DMA((N_DEV,)),
            pltpu.SemaphoreType.DMA((N_DEV,)),
        ],
    )(x, Win0, Wout0, Win1, Wout1, Win2, Wout2)
